# baseline (speedup 1.0000x reference)
import functools

import jax
import jax.numpy as jnp
from jax import lax
from jax.experimental import pallas as pl
from jax.experimental.pallas import tpu as pltpu
from jax.experimental.pallas import tpu_sc as plsc

V = 1000
VPAD = 1024
B = 1024
S = 50
N = B * S
VP = 1008

_info = plsc.get_sparse_core_info()
_NC, _NS = _info.num_cores, _info.num_subcores
NW = _NC * _NS
BPW = N // NW
SPW = B // NW
VCH = 80
NVC = BPW // VCH
NG16 = BPW // 16

_mesh = plsc.VectorSubcoreMesh(core_axis_name="c", subcore_axis_name="s")


def _wid():
  return lax.axis_index("s") * _NC + lax.axis_index("c")


def _loss_body(idx_hbm, tgt_hbm, tabflat_hbm,
               hist_hbm, s2p_hbm,
               idx_v, tgt_v, fidx_v, vals_v, hist_v, acc_v,
               sem_v):
  wid = _wid()
  base = wid * BPW
  pltpu.sync_copy(idx_hbm.at[pl.ds(base, BPW)], idx_v)
  pltpu.sync_copy(tgt_hbm.at[pl.ds(base, BPW)], tgt_v)

  zz = jnp.zeros((16,), jnp.float32)

  def zbody(i, c):
    hist_v[pl.ds(i * 16, 16)] = zz
    return c
  lax.fori_loop(0, VP // 16, zbody, 0)
  acc_v[...] = zz

  ones = jnp.ones((16,), jnp.float32)

  def hbody(i, c):
    sl = pl.ds(i * 16, 16)
    ii = idx_v[sl]
    fidx_v[sl] = ii * VPAD + tgt_v[sl]
    plsc.addupdate_scatter(hist_v, [ii], ones)
    return c
  lax.fori_loop(0, NG16, hbody, 0)

  pltpu.sync_copy(hist_v, hist_hbm.at[wid])

  def vbody(i, c):
    sl = pl.ds(i * VCH, VCH)
    pltpu.async_copy(tabflat_hbm.at[fidx_v.at[sl]], vals_v.at[sl], sem_v).wait()
    return c
  lax.fori_loop(0, NVC, vbody, 0)

  def abody(i, c):
    acc_v[...] = acc_v[...] + vals_v[pl.ds(i * 16, 16)]
    return c
  lax.fori_loop(0, NG16, abody, 0)

  pltpu.sync_copy(acc_v, s2p_hbm.at[wid])


_sc_loss = functools.partial(
    pl.kernel,
    out_type=[
        jax.ShapeDtypeStruct((NW, VP), jnp.float32),
        jax.ShapeDtypeStruct((NW, 16), jnp.float32),
    ],
    mesh=_mesh,
    compiler_params=pltpu.CompilerParams(
        needs_layout_passes=False, use_tc_tiling_on_sc=False),
    scratch_types=[
        pltpu.VMEM((BPW,), jnp.int32),
        pltpu.VMEM((BPW,), jnp.int32),
        pltpu.VMEM((BPW,), jnp.int32),
        pltpu.VMEM((BPW,), jnp.float32),
        pltpu.VMEM((VP,), jnp.float32),
        pltpu.VMEM((16,), jnp.float32),
        pltpu.SemaphoreType.DMA,
    ],
)(_loss_body)


def _gather_body(idx_hbm, tabmain_hbm, tabtail_hbm,
                 out_hbm,
                 idxp_v, main0_v, main1_v, tb0_v, tb1_v, tail_v, last2_v,
                 semg0, semg1, sems0, sems1, semt):
  wid = _wid()

  pltpu.sync_copy(idx_hbm.at[pl.ds(wid * (SPW * 56), SPW * 56)], idxp_v)

  def issue_gather(c, mbuf, tbuf, semg):
    pltpu.async_copy(tabmain_hbm.at[idxp_v.at[pl.ds(c * 56, 56)]], mbuf, semg)
    pltpu.async_copy(tabtail_hbm.at[idxp_v.at[pl.ds(c * 56, S)]], tbuf, semg)

  def wait_gather(c, mbuf, tbuf, semg):
    pltpu.make_async_copy(
        tabmain_hbm.at[idxp_v.at[pl.ds(c * 56, 56)]], mbuf, semg).wait()
    pltpu.make_async_copy(
        tabtail_hbm.at[idxp_v.at[pl.ds(c * 56, S)]], tbuf, semg).wait()

  def fill_tail(tbuf):
    for r in range(48):
      for g in range(6):
        tail_v[r, pl.ds(g * 16, 16)] = tbuf[r, pl.ds(g * 16, 16)]
      tail_v[r, pl.ds(88, 16)] = tbuf[r, pl.ds(88, 16)]

  def fill_last2(mbuf, tbuf):
    for r in range(2):
      for g in range(56):
        last2_v[r, pl.ds(g * 16, 16)] = mbuf[48 + r, pl.ds(g * 16, 16)]
      for g in range(6):
        last2_v[r, pl.ds(896 + g * 16, 16)] = tbuf[48 + r, pl.ds(g * 16, 16)]
      last2_v[r, pl.ds(984, 16)] = tbuf[48 + r, pl.ds(88, 16)]

  def issue_scatter(c, mbuf, sems):
    slab = out_hbm.at[wid * SPW + c]
    pltpu.async_copy(mbuf.at[pl.ds(0, 48), :],
                     slab.at[pl.ds(0, 48), pl.ds(0, 896)], sems)
    pltpu.async_copy(last2_v, slab.at[pl.ds(48, 2), :], semt)
    pltpu.async_copy(tail_v, slab.at[pl.ds(0, 48), pl.ds(896, V - 896)],
                     semt)

  def wait_scatter(c, mbuf, sems):
    slab = out_hbm.at[wid * SPW + c]
    pltpu.make_async_copy(mbuf.at[pl.ds(0, 48), :],
                          slab.at[pl.ds(0, 48), pl.ds(0, 896)], sems).wait()

  def wait_tail(c):
    slab = out_hbm.at[wid * SPW + c]
    pltpu.make_async_copy(last2_v, slab.at[pl.ds(48, 2), :], semt).wait()
    pltpu.make_async_copy(
        tail_v, slab.at[pl.ds(0, 48), pl.ds(896, V - 896)], semt).wait()

  def step(k, mbuf, tbuf, semg, sems, ombuf, otbuf, osemg, osems,
           first, last):
    if not first:
      wait_scatter(k - 1, ombuf, osems)
    if not last:
      issue_gather(k + 1, ombuf, otbuf, osemg)
    wait_gather(k, mbuf, tbuf, semg)
    if not first:
      wait_tail(k - 1)
    fill_tail(tbuf)
    fill_last2(mbuf, tbuf)
    issue_scatter(k, mbuf, sems)

  issue_gather(0, main0_v, tb0_v, semg0)
  step(0, main0_v, tb0_v, semg0, sems0,
       main1_v, tb1_v, semg1, sems1, True, False)

  def pbody(p, carry):
    k = 2 * p - 1
    step(k, main1_v, tb1_v, semg1, sems1,
         main0_v, tb0_v, semg0, sems0, False, False)
    step(k + 1, main0_v, tb0_v, semg0, sems0,
         main1_v, tb1_v, semg1, sems1, False, False)
    return carry
  lax.fori_loop(1, SPW // 2, pbody, 0)

  step(SPW - 1, main1_v, tb1_v, semg1, sems1,
       main0_v, tb0_v, semg0, sems0, False, True)
  wait_scatter(SPW - 1, main1_v, sems1)
  wait_tail(SPW - 1)


_sc_gather = functools.partial(
    pl.kernel,
    out_type=[
        jax.ShapeDtypeStruct((B, S, V), jnp.float32),
    ],
    mesh=_mesh,
    compiler_params=pltpu.CompilerParams(
        needs_layout_passes=False, use_tc_tiling_on_sc=True),
    scratch_types=[
        pltpu.VMEM((SPW * 56,), jnp.int32),
        pltpu.VMEM((56, 896), jnp.float32),
        pltpu.VMEM((56, 896), jnp.float32),
        pltpu.VMEM((S, 128), jnp.float32),
        pltpu.VMEM((S, 128), jnp.float32),
        pltpu.VMEM((48, V - 896), jnp.float32),
        pltpu.VMEM((2, V), jnp.float32),
        pltpu.SemaphoreType.DMA,
        pltpu.SemaphoreType.DMA,
        pltpu.SemaphoreType.DMA,
        pltpu.SemaphoreType.DMA,
        pltpu.SemaphoreType.DMA,
    ],
)(_gather_body)


def _tc_body(table_ref, hist_ref, s2p_ref, loss_ref):
  tab = table_ref[...]
  m = jnp.max(tab, axis=1, keepdims=True)
  se = jnp.sum(jnp.exp(tab - m), axis=1, keepdims=True)
  lse = m + jnp.log(se)
  cnt = jnp.sum(hist_ref[...], axis=0, keepdims=True)[:, :V]
  tot = lax.dot_general(cnt, lse, (((1,), (0,)), ((), ())),
                        preferred_element_type=jnp.float32,
                        precision=lax.Precision.HIGHEST)
  s2 = jnp.sum(s2p_ref[...])
  loss_ref[...] = (tot - s2) * (1.0 / N)


_tc_loss = pl.pallas_call(
    _tc_body,
    out_shape=jax.ShapeDtypeStruct((1, 1), jnp.float32),
)


def kernel(idx, targets, table):
  tabpad = jnp.pad(table, ((0, 0), (0, VPAD - V)))
  tabmain = table[:, :896]
  tabtail = jnp.pad(table[:, 896:], ((0, 0), (0, 128 - (V - 896))))
  idx_flat = idx.reshape(-1)
  idx_slots = jnp.pad(idx, ((0, 0), (0, 56 - S))).reshape(-1)
  (logits,) = _sc_gather(idx_slots, tabmain, tabtail)
  hist, s2p = _sc_loss(idx_flat, targets.reshape(-1), tabpad.reshape(-1))
  loss = _tc_loss(table, hist, s2p)
  return logits, loss[0, 0]


# speedup vs baseline: 1.2464x; 1.2464x over previous
import functools

import jax
import jax.numpy as jnp
from jax import lax
from jax.experimental import pallas as pl
from jax.experimental.pallas import tpu as pltpu
from jax.experimental.pallas import tpu_sc as plsc

V = 1000
N = 1024 * 50
VP = 1008

_info = plsc.get_sparse_core_info()
_NC, _NS = _info.num_cores, _info.num_subcores
NW = _NC * _NS
BPW = N // NW
RCH = 40
NRC = BPW // RCH
NG16 = BPW // 16

_mesh = plsc.VectorSubcoreMesh(core_axis_name="c", subcore_axis_name="s")


def _sc_body(idx_hbm, tgt_hbm, table_hbm,
             out_hbm, hist_hbm, s2p_hbm,
             idx_v, tgt_v, rows0_v, rows1_v, hist_v, acc_v,
             semg0, semg1, sems0, sems1):
  wid = lax.axis_index("s") * _NC + lax.axis_index("c")
  base = wid * BPW
  pltpu.sync_copy(idx_hbm.at[pl.ds(base, BPW)], idx_v)
  pltpu.sync_copy(tgt_hbm.at[pl.ds(base, BPW)], tgt_v)

  zz = jnp.zeros((16,), jnp.float32)

  def zbody(i, c):
    hist_v[pl.ds(i * 16, 16)] = zz
    return c
  lax.fori_loop(0, VP // 16, zbody, 0)
  acc_v[...] = zz

  ones = jnp.ones((16,), jnp.float32)

  def hbody(i, c):
    plsc.addupdate_scatter(hist_v, [idx_v[pl.ds(i * 16, 16)]], ones)
    return c
  lax.fori_loop(0, NG16, hbody, 0)

  pltpu.sync_copy(hist_v, hist_hbm.at[wid])

  def issue_gather(c, buf, semg):
    pltpu.async_copy(table_hbm.at[idx_v.at[pl.ds(c * RCH, RCH)]], buf, semg)

  def wait_gather(c, buf, semg):
    pltpu.make_async_copy(
        table_hbm.at[idx_v.at[pl.ds(c * RCH, RCH)]], buf, semg).wait()

  def issue_scatter(c, buf, sems):
    pltpu.async_copy(buf, out_hbm.at[pl.ds(base + c * RCH, RCH)], sems)

  def wait_scatter(c, buf, sems):
    pltpu.make_async_copy(
        buf, out_hbm.at[pl.ds(base + c * RCH, RCH)], sems).wait()

  def consume(c, buf):
    for g in range(RCH // 16):
      rr = jnp.arange(16, dtype=jnp.int32) + (g * 16)
      tt = tgt_v[pl.ds(c * RCH + g * 16, 16)]
      acc_v[...] = acc_v[...] + plsc.load_gather(buf, [rr, tt])

  def step(k, buf, semg, sems, obuf, osemg, osems, first, last):
    if not first:
      wait_scatter(k - 1, obuf, osems)
    if not last:
      issue_gather(k + 1, obuf, osemg)
    wait_gather(k, buf, semg)
    issue_scatter(k, buf, sems)
    consume(k, buf)

  issue_gather(0, rows0_v, semg0)
  step(0, rows0_v, semg0, sems0, rows1_v, semg1, sems1, True, False)

  def pbody(p, carry):
    k = 2 * p - 1
    step(k, rows1_v, semg1, sems1, rows0_v, semg0, sems0, False, False)
    step(k + 1, rows0_v, semg0, sems0, rows1_v, semg1, sems1, False, False)
    return carry
  lax.fori_loop(1, NRC // 2, pbody, 0)

  step(NRC - 1, rows1_v, semg1, sems1, rows0_v, semg0, sems0, False, True)
  wait_scatter(NRC - 1, rows1_v, sems1)

  pltpu.sync_copy(acc_v, s2p_hbm.at[wid])


_sc_gather = functools.partial(
    pl.kernel,
    out_type=[
        jax.ShapeDtypeStruct((N, V), jnp.float32),
        jax.ShapeDtypeStruct((NW, VP), jnp.float32),
        jax.ShapeDtypeStruct((NW, 16), jnp.float32),
    ],
    mesh=_mesh,
    compiler_params=pltpu.CompilerParams(
        needs_layout_passes=False, use_tc_tiling_on_sc=False),
    scratch_types=[
        pltpu.VMEM((BPW,), jnp.int32),
        pltpu.VMEM((BPW,), jnp.int32),
        pltpu.VMEM((RCH, V), jnp.float32),
        pltpu.VMEM((RCH, V), jnp.float32),
        pltpu.VMEM((VP,), jnp.float32),
        pltpu.VMEM((16,), jnp.float32),
        pltpu.SemaphoreType.DMA,
        pltpu.SemaphoreType.DMA,
        pltpu.SemaphoreType.DMA,
        pltpu.SemaphoreType.DMA,
    ],
)(_sc_body)


def _tc_body(table_ref, hist_ref, s2p_ref, loss_ref):
  tab = table_ref[...]
  m = jnp.max(tab, axis=1, keepdims=True)
  se = jnp.sum(jnp.exp(tab - m), axis=1, keepdims=True)
  lse = m + jnp.log(se)
  cnt = jnp.sum(hist_ref[...], axis=0, keepdims=True)[:, :V]
  tot = lax.dot_general(cnt, lse, (((1,), (0,)), ((), ())),
                        preferred_element_type=jnp.float32,
                        precision=lax.Precision.HIGHEST)
  s2 = jnp.sum(s2p_ref[...])
  loss_ref[...] = (tot - s2) * (1.0 / N)


_tc_loss = pl.pallas_call(
    _tc_body,
    out_shape=jax.ShapeDtypeStruct((1, 1), jnp.float32),
)


def kernel(idx, targets, table):
  b, s = idx.shape
  logits_flat, hist, s2p = _sc_gather(
      idx.reshape(-1), targets.reshape(-1), table)
  loss = _tc_loss(table, hist, s2p)
  return logits_flat.reshape(b, s, V), loss[0, 0]
